# baseline (speedup 1.0000x reference)
import functools

import jax
import jax.numpy as jnp
from jax import lax
from jax.experimental import pallas as pl
from jax.experimental.pallas import tpu as pltpu
from jax.experimental.pallas import tpu_sc as plsc

N = 8192
NSLICES = 8
NWORKERS = 32
QPW = (NSLICES * N) // NWORKERS
WPS = N // QPW
QV = QPW // 16


def _sc_kernel(qx_hbm, qy_hbm, qz_hbm, kx_hbm, ky_hbm, kz_hbm,
               idx_hbm, sums_hbm, kx_v, ky_v, kz_v,
               qx_v, qy_v, qz_v, idx_v, sum_v):
    wid = lax.axis_index("s") * 2 + lax.axis_index("c")
    sl = wid // WPS
    kbase = sl * N
    qbase = wid * QPW

    pltpu.sync_copy(kx_hbm.at[pl.ds(kbase, N)], kx_v)
    pltpu.sync_copy(ky_hbm.at[pl.ds(kbase, N)], ky_v)
    pltpu.sync_copy(kz_hbm.at[pl.ds(kbase, N)], kz_v)
    pltpu.sync_copy(qx_hbm.at[pl.ds(qbase, QPW)], qx_v)
    pltpu.sync_copy(qy_hbm.at[pl.ds(qbase, QPW)], qy_v)
    pltpu.sync_copy(qz_hbm.at[pl.ds(qbase, QPW)], qz_v)

    def qtile(qv, vsum):
        off = qv * 16
        qx = qx_v[pl.ds(off, 16)]
        qy = qy_v[pl.ds(off, 16)]
        qz = qz_v[pl.ds(off, 16)]

        def kchunk(kv, carry):
            best, bidx = carry
            base = kv * 16
            kxv = kx_v[pl.ds(base, 16)]
            kyv = ky_v[pl.ds(base, 16)]
            kzv = kz_v[pl.ds(base, 16)]
            for j in range(16):
                dx = qx - kxv[j]
                dy = qy - kyv[j]
                dz = qz - kzv[j]
                d = dx * dx + dy * dy + dz * dz
                m = d < best
                best = jnp.where(m, d, best)
                bidx = jnp.where(m, base + j, bidx)
            return best, bidx

        best0 = jnp.full((16,), jnp.inf, jnp.float32)
        bidx0 = jnp.zeros((16,), jnp.int32)
        best, bidx = lax.fori_loop(0, N // 16, kchunk, (best0, bidx0))
        idx_v[pl.ds(off, 16)] = bidx
        return vsum + best

    vsum = lax.fori_loop(0, QV, qtile, jnp.zeros((16,), jnp.float32))
    sum_v[...] = vsum

    pltpu.sync_copy(idx_v, idx_hbm.at[pl.ds(wid * QPW, QPW)])
    pltpu.sync_copy(sum_v, sums_hbm.at[pl.ds(wid * 16, 16)])


@jax.jit
def _run_sc(q, k):
    mesh = plsc.VectorSubcoreMesh(core_axis_name="c", subcore_axis_name="s")
    f = functools.partial(
        pl.kernel,
        mesh=mesh,
        out_type=[
            jax.ShapeDtypeStruct((NSLICES * N,), jnp.int32),
            jax.ShapeDtypeStruct((NWORKERS * 16,), jnp.float32),
        ],
        scratch_types=[
            pltpu.VMEM((N,), jnp.float32),
            pltpu.VMEM((N,), jnp.float32),
            pltpu.VMEM((N,), jnp.float32),
            pltpu.VMEM((QPW,), jnp.float32),
            pltpu.VMEM((QPW,), jnp.float32),
            pltpu.VMEM((QPW,), jnp.float32),
            pltpu.VMEM((QPW,), jnp.int32),
            pltpu.VMEM((16,), jnp.float32),
        ],
    )(_sc_kernel)
    return f(q[0], q[1], q[2], k[0], k[1], k[2])


def kernel(pred_points, true_points):
    q = jnp.concatenate([pred_points, true_points], axis=0).transpose(2, 0, 1)
    q = q.reshape(3, NSLICES * N)
    k = jnp.concatenate([true_points, pred_points], axis=0).transpose(2, 0, 1)
    k = k.reshape(3, NSLICES * N)
    idx_flat, sums = _run_sc(q, k)
    idx = idx_flat.reshape(NSLICES, N)
    loss = jnp.sum(sums) / jnp.float32(4 * N)
    return loss, idx[:4], idx[4:]


# speedup vs baseline: 3.0672x; 3.0672x over previous
import functools

import jax
import jax.numpy as jnp
from jax import lax
from jax.experimental import pallas as pl
from jax.experimental.pallas import tpu as pltpu
from jax.experimental.pallas import tpu_sc as plsc

N = 8192
NSLICES = 8
NS_SC = 2
NS_TC = NSLICES - NS_SC
NWORKERS = 32
QPW = (NS_SC * N) // NWORKERS
WPS = N // QPW
QV = QPW // 16

BQ = 64
LK = 128
NKB = N // LK
NQB = N // BQ
BIGI = 1 << 30


def _sc_kernel(qx_hbm, qy_hbm, qz_hbm, kx_hbm, ky_hbm, kz_hbm,
               idx_hbm, sums_hbm, kx_v, ky_v, kz_v,
               qx_v, qy_v, qz_v, idx_v, sum_v):
    wid = lax.axis_index("s") * 2 + lax.axis_index("c")
    sl = wid // WPS
    kbase = sl * N
    qbase = wid * QPW

    pltpu.sync_copy(kx_hbm.at[pl.ds(kbase, N)], kx_v)
    pltpu.sync_copy(ky_hbm.at[pl.ds(kbase, N)], ky_v)
    pltpu.sync_copy(kz_hbm.at[pl.ds(kbase, N)], kz_v)
    pltpu.sync_copy(qx_hbm.at[pl.ds(qbase, QPW)], qx_v)
    pltpu.sync_copy(qy_hbm.at[pl.ds(qbase, QPW)], qy_v)
    pltpu.sync_copy(qz_hbm.at[pl.ds(qbase, QPW)], qz_v)

    def qtile(qv, vsum):
        off = qv * 16
        qx = qx_v[pl.ds(off, 16)]
        qy = qy_v[pl.ds(off, 16)]
        qz = qz_v[pl.ds(off, 16)]

        def kchunk(kv, carry):
            best, bidx = carry
            base = kv * 16
            kxv = kx_v[pl.ds(base, 16)]
            kyv = ky_v[pl.ds(base, 16)]
            kzv = kz_v[pl.ds(base, 16)]
            for j in range(16):
                dx = qx - kxv[j]
                dy = qy - kyv[j]
                dz = qz - kzv[j]
                d = dx * dx + dy * dy + dz * dz
                m = d < best
                best = jnp.where(m, d, best)
                bidx = jnp.where(m, base + j, bidx)
            return best, bidx

        best0 = jnp.full((16,), jnp.inf, jnp.float32)
        bidx0 = jnp.zeros((16,), jnp.int32)
        best, bidx = lax.fori_loop(0, N // 16, kchunk, (best0, bidx0))
        idx_v[pl.ds(off, 16)] = bidx
        return vsum + best

    vsum = lax.fori_loop(0, QV, qtile, jnp.zeros((16,), jnp.float32))
    sum_v[...] = vsum

    pltpu.sync_copy(idx_v, idx_hbm.at[pl.ds(wid * QPW, QPW)])
    pltpu.sync_copy(sum_v, sums_hbm.at[pl.ds(wid * 16, 16)])


def _tc_kernel(q_ref, k_ref, acc_ref, idx_ref):
    s = pl.program_id(0)
    qb = pl.program_id(1)

    qxb = jnp.broadcast_to(q_ref[0, :, 0:1], (BQ, LK))
    qyb = jnp.broadcast_to(q_ref[0, :, 1:2], (BQ, LK))
    qzb = jnp.broadcast_to(q_ref[0, :, 2:3], (BQ, LK))

    def body(kb, carry):
        runvals, runkb = carry
        base = kb * LK
        kx = k_ref[0, 0:1, pl.ds(base, LK)]
        ky = k_ref[0, 1:2, pl.ds(base, LK)]
        kz = k_ref[0, 2:3, pl.ds(base, LK)]
        dx = qxb - kx
        dy = qyb - ky
        dz = qzb - kz
        d = dx * dx + dy * dy + dz * dz
        m = d < runvals
        runvals = jnp.where(m, d, runvals)
        runkb = jnp.where(m, kb, runkb)
        return runvals, runkb

    rv0 = jnp.full((BQ, LK), jnp.inf, jnp.float32)
    rk0 = jnp.zeros((BQ, LK), jnp.int32)
    runvals, runkb = lax.fori_loop(0, NKB, body, (rv0, rk0), unroll=8)

    lane = lax.broadcasted_iota(jnp.int32, (BQ, LK), 1)
    runkey = runkb * LK + lane
    bm = jnp.min(runvals, axis=1, keepdims=True)
    ridx = jnp.min(
        jnp.where(runvals == bm, runkey, BIGI), axis=1, keepdims=True
    )
    idx_ref[0, :, :] = ridx

    @pl.when(jnp.logical_and(s == 0, qb == 0))
    def _init():
        acc_ref[0, 0] = 0.0

    acc_ref[0, 0] += jnp.sum(bm)


@jax.jit
def _run(q_tc, k_tc, q_sc, k_sc):
    mesh = plsc.VectorSubcoreMesh(core_axis_name="c", subcore_axis_name="s")
    sc = functools.partial(
        pl.kernel,
        mesh=mesh,
        out_type=[
            jax.ShapeDtypeStruct((NS_SC * N,), jnp.int32),
            jax.ShapeDtypeStruct((NWORKERS * 16,), jnp.float32),
        ],
        scratch_types=[
            pltpu.VMEM((N,), jnp.float32),
            pltpu.VMEM((N,), jnp.float32),
            pltpu.VMEM((N,), jnp.float32),
            pltpu.VMEM((QPW,), jnp.float32),
            pltpu.VMEM((QPW,), jnp.float32),
            pltpu.VMEM((QPW,), jnp.float32),
            pltpu.VMEM((QPW,), jnp.int32),
            pltpu.VMEM((16,), jnp.float32),
        ],
    )(_sc_kernel)
    idx_sc, sums_sc = sc(q_sc[0], q_sc[1], q_sc[2], k_sc[0], k_sc[1], k_sc[2])

    acc_tc, idx_tc = pl.pallas_call(
        _tc_kernel,
        grid=(NS_TC, NQB),
        in_specs=[
            pl.BlockSpec((1, BQ, 3), lambda s, qb: (s, qb, 0)),
            pl.BlockSpec((1, 3, N), lambda s, qb: (s, 0, 0)),
        ],
        out_specs=[
            pl.BlockSpec((1, 1), lambda s, qb: (0, 0), memory_space=pltpu.SMEM),
            pl.BlockSpec((1, BQ, 1), lambda s, qb: (s, qb, 0)),
        ],
        out_shape=[
            jax.ShapeDtypeStruct((1, 1), jnp.float32),
            jax.ShapeDtypeStruct((NS_TC, N, 1), jnp.int32),
        ],
    )(q_tc, k_tc)

    idx = jnp.concatenate(
        [idx_tc.reshape(NS_TC, N), idx_sc.reshape(NS_SC, N)], axis=0
    )
    loss = (acc_tc[0, 0] + jnp.sum(sums_sc)) / jnp.float32(4 * N)
    return loss, idx


def kernel(pred_points, true_points):
    q_all = jnp.concatenate([pred_points, true_points], axis=0)
    k_all = jnp.concatenate([true_points, pred_points], axis=0)
    q_tc = q_all[:NS_TC]
    k_tc = k_all[:NS_TC].transpose(0, 2, 1)
    q_sc = q_all[NS_TC:].transpose(2, 0, 1).reshape(3, NS_SC * N)
    k_sc = k_all[NS_TC:].transpose(2, 0, 1).reshape(3, NS_SC * N)
    loss, idx = _run(q_tc, k_tc, q_sc, k_sc)
    return loss, idx[:4], idx[4:]


# speedup vs baseline: 3.2568x; 1.0618x over previous
import functools

import jax
import jax.numpy as jnp
from jax import lax
from jax.experimental import pallas as pl
from jax.experimental.pallas import tpu as pltpu
from jax.experimental.pallas import tpu_sc as plsc

N = 8192
NSLICES = 8
Q_TC = 5760
Q_SC = N - Q_TC
NWORKERS = 32
WPS = NWORKERS // NSLICES
QPW = Q_SC // WPS
QV = QPW // 16

BQ = 64
LK = 128
NKB = N // LK
NQB_TC = Q_TC // BQ
BIGI = 1 << 30


def _sc_kernel(qx_hbm, qy_hbm, qz_hbm, kx_hbm, ky_hbm, kz_hbm,
               idx_hbm, sums_hbm, kx_v, ky_v, kz_v,
               qx_v, qy_v, qz_v, idx_v, sum_v):
    wid = lax.axis_index("s") * 2 + lax.axis_index("c")
    sl = wid // WPS
    part = wid % WPS
    kbase = sl * N
    qbase = sl * N + Q_TC + part * QPW

    pltpu.sync_copy(kx_hbm.at[pl.ds(kbase, N)], kx_v)
    pltpu.sync_copy(ky_hbm.at[pl.ds(kbase, N)], ky_v)
    pltpu.sync_copy(kz_hbm.at[pl.ds(kbase, N)], kz_v)
    pltpu.sync_copy(qx_hbm.at[pl.ds(qbase, QPW)], qx_v)
    pltpu.sync_copy(qy_hbm.at[pl.ds(qbase, QPW)], qy_v)
    pltpu.sync_copy(qz_hbm.at[pl.ds(qbase, QPW)], qz_v)

    def qtile(qv, vsum):
        off = qv * 16
        qx = qx_v[pl.ds(off, 16)]
        qy = qy_v[pl.ds(off, 16)]
        qz = qz_v[pl.ds(off, 16)]

        def kchunk(kv, carry):
            best, bidx = carry
            base = kv * 16
            kxv = kx_v[pl.ds(base, 16)]
            kyv = ky_v[pl.ds(base, 16)]
            kzv = kz_v[pl.ds(base, 16)]
            for j in range(16):
                dx = qx - kxv[j]
                dy = qy - kyv[j]
                dz = qz - kzv[j]
                d = dx * dx + dy * dy + dz * dz
                m = d < best
                best = jnp.where(m, d, best)
                bidx = jnp.where(m, base + j, bidx)
            return best, bidx

        best0 = jnp.full((16,), jnp.inf, jnp.float32)
        bidx0 = jnp.zeros((16,), jnp.int32)
        best, bidx = lax.fori_loop(0, N // 16, kchunk, (best0, bidx0))
        idx_v[pl.ds(off, 16)] = bidx
        return vsum + best

    vsum = lax.fori_loop(0, QV, qtile, jnp.zeros((16,), jnp.float32))
    sum_v[...] = vsum

    pltpu.sync_copy(idx_v, idx_hbm.at[pl.ds(wid * QPW, QPW)])
    pltpu.sync_copy(sum_v, sums_hbm.at[pl.ds(wid * 16, 16)])


def _tc_kernel(q_ref, k_ref, acc_ref, idx_ref):
    s = pl.program_id(0)
    qb = pl.program_id(1)

    qxb = jnp.broadcast_to(q_ref[0, :, 0:1], (BQ, LK))
    qyb = jnp.broadcast_to(q_ref[0, :, 1:2], (BQ, LK))
    qzb = jnp.broadcast_to(q_ref[0, :, 2:3], (BQ, LK))

    def body(kb, carry):
        runvals, runkb = carry
        base = kb * LK
        kx = k_ref[0, 0:1, pl.ds(base, LK)]
        ky = k_ref[0, 1:2, pl.ds(base, LK)]
        kz = k_ref[0, 2:3, pl.ds(base, LK)]
        dx = qxb - kx
        dy = qyb - ky
        dz = qzb - kz
        d = dx * dx + dy * dy + dz * dz
        m = d < runvals
        runvals = jnp.where(m, d, runvals)
        runkb = jnp.where(m, kb, runkb)
        return runvals, runkb

    rv0 = jnp.full((BQ, LK), jnp.inf, jnp.float32)
    rk0 = jnp.zeros((BQ, LK), jnp.int32)
    runvals, runkb = lax.fori_loop(0, NKB, body, (rv0, rk0), unroll=8)

    lane = lax.broadcasted_iota(jnp.int32, (BQ, LK), 1)
    runkey = runkb * LK + lane
    bm = jnp.min(runvals, axis=1, keepdims=True)
    ridx = jnp.min(
        jnp.where(runvals == bm, runkey, BIGI), axis=1, keepdims=True
    )
    idx_ref[0, :, :] = ridx

    @pl.when(jnp.logical_and(s == 0, qb == 0))
    def _init():
        acc_ref[0, 0] = 0.0

    acc_ref[0, 0] += jnp.sum(bm)


@jax.jit
def _run(q_tc, k_tc, q_sc, k_sc):
    mesh = plsc.VectorSubcoreMesh(core_axis_name="c", subcore_axis_name="s")
    sc = functools.partial(
        pl.kernel,
        mesh=mesh,
        out_type=[
            jax.ShapeDtypeStruct((NSLICES * Q_SC,), jnp.int32),
            jax.ShapeDtypeStruct((NWORKERS * 16,), jnp.float32),
        ],
        scratch_types=[
            pltpu.VMEM((N,), jnp.float32),
            pltpu.VMEM((N,), jnp.float32),
            pltpu.VMEM((N,), jnp.float32),
            pltpu.VMEM((QPW,), jnp.float32),
            pltpu.VMEM((QPW,), jnp.float32),
            pltpu.VMEM((QPW,), jnp.float32),
            pltpu.VMEM((QPW,), jnp.int32),
            pltpu.VMEM((16,), jnp.float32),
        ],
    )(_sc_kernel)
    idx_sc, sums_sc = sc(q_sc[0], q_sc[1], q_sc[2], k_sc[0], k_sc[1], k_sc[2])

    acc_tc, idx_tc = pl.pallas_call(
        _tc_kernel,
        grid=(NSLICES, NQB_TC),
        in_specs=[
            pl.BlockSpec((1, BQ, 3), lambda s, qb: (s, qb, 0)),
            pl.BlockSpec((1, 3, N), lambda s, qb: (s, 0, 0)),
        ],
        out_specs=[
            pl.BlockSpec((1, 1), lambda s, qb: (0, 0), memory_space=pltpu.SMEM),
            pl.BlockSpec((1, BQ, 1), lambda s, qb: (s, qb, 0)),
        ],
        out_shape=[
            jax.ShapeDtypeStruct((1, 1), jnp.float32),
            jax.ShapeDtypeStruct((NSLICES, Q_TC, 1), jnp.int32),
        ],
    )(q_tc, k_tc)

    idx = jnp.concatenate(
        [idx_tc.reshape(NSLICES, Q_TC), idx_sc.reshape(NSLICES, Q_SC)], axis=1
    )
    loss = (acc_tc[0, 0] + jnp.sum(sums_sc)) / jnp.float32(4 * N)
    return loss, idx


def kernel(pred_points, true_points):
    q_all = jnp.concatenate([pred_points, true_points], axis=0)
    k_all = jnp.concatenate([true_points, pred_points], axis=0)
    k_tc = k_all.transpose(0, 2, 1)
    q_sc = q_all.transpose(2, 0, 1).reshape(3, NSLICES * N)
    k_sc = k_all.transpose(2, 0, 1).reshape(3, NSLICES * N)
    loss, idx = _run(q_all, k_tc, q_sc, k_sc)
    return loss, idx[:4], idx[4:]
